# baseline (speedup 1.0000x reference)
import functools

import jax
import jax.numpy as jnp
from jax.experimental import pallas as pl
from jax.experimental.pallas import tpu as pltpu

BLK = 1024


def _leaky(v):
    return jnp.where(v >= 0, v, 0.01 * v)


def _gcn_kernel(x_ref, adj_ref, gc1w_ref, gc1b_ref, w11_ref, b11_ref,
                w12_ref, b12_ref, w21_ref, b21_ref, w22_ref, b22_ref,
                gc6w_ref, gc6b_ref, z_out, xout_out, support_scr, z_scr,
                m_scr, *, nblocks):
    i = pl.program_id(0)

    @pl.when(i == 0)
    def _():
        support_scr[:] = jnp.dot(x_ref[:], gc1w_ref[:],
                                 preferred_element_type=jnp.float32
                                 ).astype(jnp.bfloat16)
        m_scr[:] = jnp.zeros_like(m_scr)

    h = jnp.dot(adj_ref[:].astype(jnp.bfloat16), support_scr[:],
                preferred_element_type=jnp.float32) + gc1b_ref[:]
    h = _leaky(h)
    h = _leaky(jnp.dot(h, w11_ref[:], preferred_element_type=jnp.float32)
               + b11_ref[:])
    z = jnp.dot(h, w12_ref[:], preferred_element_type=jnp.float32) + b12_ref[:]
    z_out[:] = z
    z_scr[pl.ds(i * z.shape[0], z.shape[0]), :] = z
    d = _leaky(jnp.dot(z, w21_ref[:], preferred_element_type=jnp.float32)
               + b21_ref[:])
    d = _leaky(jnp.dot(d, w22_ref[:], preferred_element_type=jnp.float32)
               + b22_ref[:])
    g = jnp.dot(d, gc6w_ref[:], preferred_element_type=jnp.float32)
    m_scr[:] += jnp.dot(z.T, g, preferred_element_type=jnp.float32)

    @pl.when(i == nblocks - 1)
    def _():
        xout_out[:] = jnp.dot(z_scr[:], m_scr[:],
                              preferred_element_type=jnp.float32) + gc6b_ref[:]


@jax.jit
def kernel(x, adj, gc1_W, gc1_b, c1_W1, c1_b1, c1_W2, c1_b2,
           c2_W1, c2_b1, c2_W2, c2_b2, gc6_W, gc6_b):
    n, in_dim = x.shape
    h0 = gc1_W.shape[1]
    h1 = c1_W1.shape[1]
    h2 = c1_W2.shape[1]
    nblocks = n // BLK

    full = lambda s: pl.BlockSpec(s, lambda i: (0, 0))

    z, x_out = pl.pallas_call(
        functools.partial(_gcn_kernel, nblocks=nblocks),
        grid=(nblocks,),
        in_specs=[
            full((n, in_dim)),
            pl.BlockSpec((BLK, n), lambda i: (i, 0)),
            full((in_dim, h0)),
            full((1, h0)),
            full((h0, h1)),
            full((1, h1)),
            full((h1, h2)),
            full((1, h2)),
            full((h2, h1)),
            full((1, h1)),
            full((h1, h0)),
            full((1, h0)),
            full((h0, in_dim)),
            full((1, in_dim)),
        ],
        out_specs=[
            pl.BlockSpec((BLK, h2), lambda i: (i, 0)),
            full((n, in_dim)),
        ],
        out_shape=[
            jax.ShapeDtypeStruct((n, h2), jnp.float32),
            jax.ShapeDtypeStruct((n, in_dim), jnp.float32),
        ],
        scratch_shapes=[
            pltpu.VMEM((n, h0), jnp.bfloat16),
            pltpu.VMEM((n, h2), jnp.float32),
            pltpu.VMEM((h2, in_dim), jnp.float32),
        ],
        compiler_params=pltpu.CompilerParams(
            dimension_semantics=("arbitrary",)),
    )(x, adj, gc1_W, gc1_b.reshape(1, -1), c1_W1, c1_b1.reshape(1, -1),
      c1_W2, c1_b2.reshape(1, -1), c2_W1, c2_b1.reshape(1, -1),
      c2_W2, c2_b2.reshape(1, -1), gc6_W, gc6_b.reshape(1, -1))

    return (x_out, z)


# speedup vs baseline: 1.0032x; 1.0032x over previous
import functools

import jax
import jax.numpy as jnp
from jax.experimental import pallas as pl
from jax.experimental.pallas import tpu as pltpu

BLK = 1024


def _leaky(v):
    return jnp.where(v >= 0, v, 0.01 * v)


def _gcn_kernel(x_ref, adj_ref, gc1w_ref, gc1b_ref, w11_ref, b11_ref,
                w12_ref, b12_ref, w21_ref, b21_ref, w22_ref, b22_ref,
                gc6w_ref, gc6b_ref, z_out, xout_out, support_scr, z_scr,
                m_scr, *, nblocks):
    i = pl.program_id(0)

    @pl.when(i == 0)
    def _():
        support_scr[:] = jnp.dot(x_ref[:], gc1w_ref[:],
                                 preferred_element_type=jnp.float32)
        m_scr[:] = jnp.zeros_like(m_scr)

    h = jnp.dot(adj_ref[:], support_scr[:],
                preferred_element_type=jnp.float32) + gc1b_ref[:]
    h = _leaky(h)
    h = _leaky(jnp.dot(h, w11_ref[:], preferred_element_type=jnp.float32)
               + b11_ref[:])
    z = jnp.dot(h, w12_ref[:], preferred_element_type=jnp.float32) + b12_ref[:]
    z_out[:] = z
    z_scr[pl.ds(i * z.shape[0], z.shape[0]), :] = z
    d = _leaky(jnp.dot(z, w21_ref[:], preferred_element_type=jnp.float32)
               + b21_ref[:])
    d = _leaky(jnp.dot(d, w22_ref[:], preferred_element_type=jnp.float32)
               + b22_ref[:])
    g = jnp.dot(d, gc6w_ref[:], preferred_element_type=jnp.float32)
    m_scr[:] += jnp.dot(z.T, g, preferred_element_type=jnp.float32)

    @pl.when(i == nblocks - 1)
    def _():
        xout_out[:] = jnp.dot(z_scr[:], m_scr[:],
                              preferred_element_type=jnp.float32) + gc6b_ref[:]


@jax.jit
def kernel(x, adj, gc1_W, gc1_b, c1_W1, c1_b1, c1_W2, c1_b2,
           c2_W1, c2_b1, c2_W2, c2_b2, gc6_W, gc6_b):
    n, in_dim = x.shape
    h0 = gc1_W.shape[1]
    h1 = c1_W1.shape[1]
    h2 = c1_W2.shape[1]
    nblocks = n // BLK

    full = lambda s: pl.BlockSpec(s, lambda i: (0, 0))

    z, x_out = pl.pallas_call(
        functools.partial(_gcn_kernel, nblocks=nblocks),
        grid=(nblocks,),
        in_specs=[
            full((n, in_dim)),
            pl.BlockSpec((BLK, n), lambda i: (i, 0)),
            full((in_dim, h0)),
            full((1, h0)),
            full((h0, h1)),
            full((1, h1)),
            full((h1, h2)),
            full((1, h2)),
            full((h2, h1)),
            full((1, h1)),
            full((h1, h0)),
            full((1, h0)),
            full((h0, in_dim)),
            full((1, in_dim)),
        ],
        out_specs=[
            pl.BlockSpec((BLK, h2), lambda i: (i, 0)),
            full((n, in_dim)),
        ],
        out_shape=[
            jax.ShapeDtypeStruct((n, h2), jnp.float32),
            jax.ShapeDtypeStruct((n, in_dim), jnp.float32),
        ],
        scratch_shapes=[
            pltpu.VMEM((n, h0), jnp.float32),
            pltpu.VMEM((n, h2), jnp.float32),
            pltpu.VMEM((h2, in_dim), jnp.float32),
        ],
        compiler_params=pltpu.CompilerParams(
            dimension_semantics=("arbitrary",)),
    )(x, adj, gc1_W, gc1_b.reshape(1, -1), c1_W1, c1_b1.reshape(1, -1),
      c1_W2, c1_b2.reshape(1, -1), c2_W1, c2_b1.reshape(1, -1),
      c2_W2, c2_b2.reshape(1, -1), gc6_W, gc6_b.reshape(1, -1))

    return (x_out, z)
